# Initial kernel scaffold; baseline (speedup 1.0000x reference)
import jax
import jax.numpy as jnp
from jax.experimental import pallas as pl


def kernel(x, seg, tok_table, pos_table, seg_table, gamma, beta):
    raise NotImplementedError("write your pallas kernel here")


# speedup vs baseline: 4.7747x; 4.7747x over previous
import functools

import jax
import jax.numpy as jnp
from jax import lax
from jax.experimental import pallas as pl
from jax.experimental.pallas import tpu as pltpu
from jax.experimental.pallas import tpu_sc as plsc

_NC = 2
_NS = 16
_NW = _NC * _NS
_L = 16

_CW = 128
_EPS = 1e-5


def _rsqrt_newton(v):
    i = lax.bitcast_convert_type(v, jnp.int32)
    y = lax.bitcast_convert_type(jnp.int32(0x5F3759DF) - (i >> 1),
                                 jnp.float32)
    for _ in range(3):
        y = y * (1.5 - 0.5 * v * y * y)
    return y


_DNUMS = lax.GatherDimensionNumbers(
    offset_dims=(), collapsed_slice_dims=(0,), start_index_map=(0,))


def _permute(v, idx):
    return lax.gather(v, idx[:, None], _DNUMS, slice_sizes=(1,),
                      mode=lax.GatherScatterMode.PROMISE_IN_BOUNDS)


def _lane_sum(v, bfly_idx):
    for idx in bfly_idx:
        v = v + _permute(v, idx)
    return v


def _make_kernel(n_chunks, seq, d_model):
    per_w = n_chunks // _NW
    nk = d_model // _L

    mesh = plsc.VectorSubcoreMesh(core_axis_name="c", subcore_axis_name="s")

    @functools.partial(
        pl.kernel,
        out_type=jax.ShapeDtypeStruct((_NW, per_w, _CW, d_model),
                                      jnp.float32),
        mesh=mesh,
        scratch_types=[
            pltpu.VMEM((per_w, _CW), jnp.int32),
            pltpu.VMEM((per_w, _CW), jnp.int32),
            pltpu.VMEM((seq, d_model), jnp.float32),
            pltpu.VMEM((2, d_model), jnp.float32),
            pltpu.VMEM((d_model,), jnp.float32),
            pltpu.VMEM((d_model,), jnp.float32),
            pltpu.VMEM((_CW, d_model), jnp.float32),
            pltpu.SemaphoreType.DMA,
        ],
    )
    def emb_kernel(x_hbm, seg_hbm, tok_hbm, pos_hbm, segt_hbm, g_hbm, b_hbm,
                   out_hbm, idx_v, sgv_v, pos_v, segt_v, g_v, b_v, rows_v,
                   gsem):
        wid = lax.axis_index("s") * _NC + lax.axis_index("c")
        base = wid * per_w

        pltpu.sync_copy(x_hbm.at[wid], idx_v)
        pltpu.sync_copy(seg_hbm.at[wid], sgv_v)
        pltpu.sync_copy(pos_hbm, pos_v)
        pltpu.sync_copy(segt_hbm, segt_v)
        pltpu.sync_copy(g_hbm, g_v)
        pltpu.sync_copy(b_hbm, b_v)

        g_k = [g_v[pl.ds(k * _L, _L)] for k in range(nk)]
        b_k = [b_v[pl.ds(k * _L, _L)] for k in range(nk)]
        s0_k = [segt_v[0, pl.ds(k * _L, _L)] for k in range(nk)]
        d_k = [segt_v[1, pl.ds(k * _L, _L)] - s0_k[k] for k in range(nk)]
        inv_d = jnp.float32(1.0 / d_model)
        lane = lax.iota(jnp.int32, _L)
        bfly_idx = [lane ^ d for d in (1, 2, 4, 8)]

        def chunk_body(j, _):
            pltpu.async_copy(tok_hbm.at[idx_v.at[j]], rows_v, gsem).wait()
            f0 = (base + j) * _CW

            def group_body(g, _):
                i0 = g * _L
                sv16 = sgv_v[j, pl.ds(i0, _L)].astype(jnp.float32)
                for r in range(_L):
                    i = i0 + r
                    p = lax.rem(f0 + i, seq)
                    sv = jnp.full((_L,), sv16[r], jnp.float32)
                    h = []
                    acc = None
                    acc2 = None
                    for k in range(nk):
                        hk = (rows_v[i, pl.ds(k * _L, _L)]
                              + pos_v[p, pl.ds(k * _L, _L)]
                              + (s0_k[k] + sv * d_k[k]))
                        h.append(hk)
                        acc = hk if acc is None else acc + hk
                        acc2 = hk * hk if acc2 is None else acc2 + hk * hk
                    mv = _lane_sum(acc, bfly_idx) * inv_d
                    ex2 = _lane_sum(acc2, bfly_idx) * inv_d
                    inv_std = _rsqrt_newton(ex2 - mv * mv + _EPS)
                    for k in range(nk):
                        rows_v[i, pl.ds(k * _L, _L)] = (
                            (h[k] - mv) * inv_std * g_k[k] + b_k[k])
                return ()

            lax.fori_loop(0, _CW // _L, group_body, (), unroll=False)
            pltpu.sync_copy(rows_v, out_hbm.at[wid, j])
            return ()

        lax.fori_loop(0, per_w, chunk_body, (), unroll=False)

    return emb_kernel


def kernel(x, seg, tok_table, pos_table, seg_table, gamma, beta):
    b, s = x.shape
    v, d = tok_table.shape
    total = b * s
    n_chunks = total // _CW
    per_w = n_chunks // _NW
    xf = x.reshape(_NW, per_w, _CW).astype(jnp.int32)
    sf = seg.reshape(_NW, per_w, _CW).astype(jnp.int32)
    out = _make_kernel(n_chunks, s, d)(
        xf, sf, tok_table, pos_table, seg_table, gamma, beta)
    return out.reshape(b, s, d)


# speedup vs baseline: 6.5699x; 1.3760x over previous
import functools

import jax
import jax.numpy as jnp
from jax import lax
from jax.experimental import pallas as pl
from jax.experimental.pallas import tpu as pltpu
from jax.experimental.pallas import tpu_sc as plsc

_NC = 2
_NS = 16
_NW = _NC * _NS
_L = 16

_CW = 128
_NBUF = 3
_EPS = 1e-5


def _rsqrt_newton(v):
    i = lax.bitcast_convert_type(v, jnp.int32)
    y = lax.bitcast_convert_type(jnp.int32(0x5F3759DF) - (i >> 1),
                                 jnp.float32)
    for _ in range(2):
        y = y * (1.5 - 0.5 * v * y * y)
    return y


_DNUMS = lax.GatherDimensionNumbers(
    offset_dims=(), collapsed_slice_dims=(0,), start_index_map=(0,))


def _permute(v, idx):
    return lax.gather(v, idx[:, None], _DNUMS, slice_sizes=(1,),
                      mode=lax.GatherScatterMode.PROMISE_IN_BOUNDS)


def _lane_sum(v, bfly_idx):
    for idx in bfly_idx:
        v = v + _permute(v, idx)
    return v


def _make_kernel(n_chunks, seq, d_model):
    per_w = n_chunks // _NW
    nk = d_model // _L

    mesh = plsc.VectorSubcoreMesh(core_axis_name="c", subcore_axis_name="s")

    @functools.partial(
        pl.kernel,
        out_type=jax.ShapeDtypeStruct((_NW, per_w, _CW, d_model),
                                      jnp.float32),
        mesh=mesh,
        scratch_types=[
            pltpu.VMEM((per_w, _CW), jnp.int32),
            pltpu.VMEM((per_w, _CW), jnp.int32),
            pltpu.VMEM((2 * seq, d_model), jnp.float32),
            pltpu.VMEM((d_model,), jnp.float32),
            pltpu.VMEM((d_model,), jnp.float32),
            pltpu.VMEM((_CW, d_model), jnp.float32),
            pltpu.VMEM((_CW, d_model), jnp.float32),
            pltpu.VMEM((_CW, d_model), jnp.float32),
            pltpu.SemaphoreType.DMA,
            pltpu.SemaphoreType.DMA,
            pltpu.SemaphoreType.DMA,
            pltpu.SemaphoreType.DMA,
            pltpu.SemaphoreType.DMA,
            pltpu.SemaphoreType.DMA,
        ],
    )
    def emb_kernel(x_hbm, seg_hbm, tok_hbm, pos_hbm, segt_hbm, g_hbm, b_hbm,
                   out_hbm, idx_v, sgv_v, comb_v, g_v, b_v,
                   rows0, rows1, rows2, gs0, gs1, gs2, os0, os1, os2):
        rows = (rows0, rows1, rows2)
        gsem = (gs0, gs1, gs2)
        osem = (os0, os1, os2)
        wid = lax.axis_index("s") * _NC + lax.axis_index("c")
        base = wid * per_w

        pltpu.sync_copy(x_hbm.at[wid], idx_v)

        def start_gather(c, b):
            pltpu.async_copy(tok_hbm.at[idx_v.at[c]], rows[b], gsem[b])

        def wait_gather(c, b):
            pltpu.make_async_copy(
                tok_hbm.at[idx_v.at[c]], rows[b], gsem[b]).wait()

        def start_out(c, b):
            pltpu.async_copy(rows[b], out_hbm.at[wid, c], osem[b])

        def wait_out(c, b):
            pltpu.make_async_copy(
                rows[b], out_hbm.at[wid, c], osem[b]).wait()

        start_gather(0, 0)
        start_gather(1, 1)

        pltpu.sync_copy(seg_hbm.at[wid], sgv_v)
        pltpu.sync_copy(pos_hbm, comb_v.at[pl.ds(0, seq)])
        pltpu.sync_copy(pos_hbm, comb_v.at[pl.ds(seq, seq)])
        pltpu.sync_copy(g_hbm, g_v)
        pltpu.sync_copy(b_hbm, b_v)

        segt_stage = rows2
        pltpu.sync_copy(segt_hbm, segt_stage.at[pl.ds(0, 2)])
        seg_k = [[segt_stage[s, pl.ds(k * _L, _L)] for k in range(nk)]
                 for s in range(2)]

        def comb_body(p, _):
            for k in range(nk):
                sl = pl.ds(k * _L, _L)
                comb_v[p, sl] = comb_v[p, sl] + seg_k[0][k]
                comb_v[seq + p, sl] = comb_v[seq + p, sl] + seg_k[1][k]
            return ()

        lax.fori_loop(0, seq, comb_body, (), unroll=False)

        g_k = [g_v[pl.ds(k * _L, _L)] for k in range(nk)]
        b_k = [b_v[pl.ds(k * _L, _L)] for k in range(nk)]
        inv_d = jnp.float32(1.0 / d_model)
        lane = lax.iota(jnp.int32, _L)
        bfly_idx = [lane ^ d for d in (1, 2, 4, 8)]

        def compute_chunk(c, buf):
            f0 = (base + c) * _CW

            def group_body(g, _):
                i0 = g * _L
                sv16 = sgv_v[c, pl.ds(i0, _L)]
                for r in range(_L):
                    i = i0 + r
                    p = lax.rem(f0 + i, seq) + sv16[r] * seq
                    h = []
                    acc = None
                    acc2 = None
                    for k in range(nk):
                        hk = (buf[i, pl.ds(k * _L, _L)]
                              + comb_v[p, pl.ds(k * _L, _L)])
                        h.append(hk)
                        acc = hk if acc is None else acc + hk
                        acc2 = hk * hk if acc2 is None else acc2 + hk * hk
                    mv = _lane_sum(acc, bfly_idx) * inv_d
                    ex2 = _lane_sum(acc2, bfly_idx) * inv_d
                    inv_std = _rsqrt_newton(ex2 - mv * mv + _EPS)
                    for k in range(nk):
                        buf[i, pl.ds(k * _L, _L)] = (
                            (h[k] - mv) * inv_std * g_k[k] + b_k[k])
                return ()

            lax.fori_loop(0, _CW // _L, group_body, (), unroll=False)

        def slot(c, b, with_gather):
            wait_gather(c, b)
            compute_chunk(c, rows[b])
            start_out(c, b)
            if with_gather:
                nb = (b + 2) % _NBUF

                @pl.when(c >= 1)
                def _():
                    wait_out(c - 1, nb)

                start_gather(c + 2, nb)

        def pipe_body(t, _):
            c = t * _NBUF
            for r in range(_NBUF):
                slot(c + r, r, True)
            return ()

        n_full = (per_w - 2) // _NBUF
        lax.fori_loop(0, n_full, pipe_body, (), unroll=False)
        for r in range(per_w - _NBUF * n_full):
            slot(_NBUF * n_full + r, r, False)
        for c in (per_w - 3, per_w - 2, per_w - 1):
            wait_out(c, c % _NBUF)

    return emb_kernel


def kernel(x, seg, tok_table, pos_table, seg_table, gamma, beta):
    b, s = x.shape
    v, d = tok_table.shape
    total = b * s
    n_chunks = total // _CW
    per_w = n_chunks // _NW
    xf = x.reshape(_NW, per_w, _CW).astype(jnp.int32)
    sf = seg.reshape(_NW, per_w, _CW).astype(jnp.int32)
    out = _make_kernel(n_chunks, s, d)(
        xf, sf, tok_table, pos_table, seg_table, gamma, beta)
    return out.reshape(b, s, d)


# speedup vs baseline: 7.8384x; 1.1931x over previous
import functools

import jax
import jax.numpy as jnp
from jax import lax
from jax.experimental import pallas as pl
from jax.experimental.pallas import tpu as pltpu
from jax.experimental.pallas import tpu_sc as plsc

_NC = 2
_NS = 16
_NW = _NC * _NS
_L = 16

_CW = 128
_NBUF = 3
_EPS = 1e-5
_NSLICE = 2
_RB = 8


def _make_gather(n_chunks, d_model):
    per_w = n_chunks // _NW

    mesh = plsc.VectorSubcoreMesh(core_axis_name="c", subcore_axis_name="s")

    @functools.partial(
        pl.kernel,
        out_type=jax.ShapeDtypeStruct((_NW, per_w, _CW, d_model),
                                      jnp.float32),
        mesh=mesh,
        scratch_types=[
            pltpu.VMEM((per_w, _CW), jnp.int32),
            pltpu.VMEM((_CW, d_model), jnp.float32),
            pltpu.VMEM((_CW, d_model), jnp.float32),
            pltpu.VMEM((_CW, d_model), jnp.float32),
            pltpu.SemaphoreType.DMA,
            pltpu.SemaphoreType.DMA,
            pltpu.SemaphoreType.DMA,
            pltpu.SemaphoreType.DMA,
            pltpu.SemaphoreType.DMA,
            pltpu.SemaphoreType.DMA,
        ],
    )
    def gather_kernel(x_hbm, tok_hbm, out_hbm, idx_v,
                      rows0, rows1, rows2, gs0, gs1, gs2, os0, os1, os2):
        rows = (rows0, rows1, rows2)
        gsem = (gs0, gs1, gs2)
        osem = (os0, os1, os2)
        wid = lax.axis_index("s") * _NC + lax.axis_index("c")

        pltpu.sync_copy(x_hbm.at[wid], idx_v)

        def start_gather(c, b):
            pltpu.async_copy(tok_hbm.at[idx_v.at[c]], rows[b], gsem[b])

        def wait_gather(c, b):
            pltpu.make_async_copy(
                tok_hbm.at[idx_v.at[c]], rows[b], gsem[b]).wait()

        def start_out(c, b):
            pltpu.async_copy(rows[b], out_hbm.at[wid, c], osem[b])

        def wait_out(c, b):
            pltpu.make_async_copy(
                rows[b], out_hbm.at[wid, c], osem[b]).wait()

        start_gather(0, 0)
        start_gather(1, 1)

        def slot(c, b, with_gather):
            wait_gather(c, b)
            start_out(c, b)
            if with_gather:
                nb = (b + 2) % _NBUF

                @pl.when(c >= 1)
                def _():
                    wait_out(c - 1, nb)

                start_gather(c + 2, nb)

        def pipe_body(t, _):
            c = t * _NBUF
            for r in range(_NBUF):
                slot(c + r, r, True)
            return ()

        n_full = (per_w - 2) // _NBUF
        lax.fori_loop(0, n_full, pipe_body, (), unroll=False)
        for c in range(_NBUF * n_full, per_w):
            slot(c, c % _NBUF, c <= per_w - 3)
        for c in (per_w - 3, per_w - 2, per_w - 1):
            wait_out(c, c % _NBUF)

    return gather_kernel


def _ln_block(tok_ref, seg_ref, comb_a_ref, comb_d_ref, g_ref, b_ref,
              out_ref):
    t = tok_ref[...]
    s = seg_ref[...][..., None]
    h = t + comb_a_ref[...][None] + s * comb_d_ref[...][None]
    mean = jnp.mean(h, axis=-1, keepdims=True)
    var = jnp.mean(jnp.square(h - mean), axis=-1, keepdims=True)
    norm = (h - mean) * lax.rsqrt(var + _EPS)
    out_ref[...] = norm * g_ref[...][0][None, None] + b_ref[...][0][None, None]


def _make_ln(nb, seq, d_model):
    grid = (nb // _RB,)
    return pl.pallas_call(
        _ln_block,
        grid=grid,
        in_specs=[
            pl.BlockSpec((_RB, seq, d_model), lambda i: (i, 0, 0)),
            pl.BlockSpec((_RB, seq), lambda i: (i, 0)),
            pl.BlockSpec((seq, d_model), lambda i: (0, 0)),
            pl.BlockSpec((seq, d_model), lambda i: (0, 0)),
            pl.BlockSpec((8, d_model), lambda i: (0, 0)),
            pl.BlockSpec((8, d_model), lambda i: (0, 0)),
        ],
        out_specs=pl.BlockSpec((_RB, seq, d_model), lambda i: (i, 0, 0)),
        out_shape=jax.ShapeDtypeStruct((nb, seq, d_model), jnp.float32),
    )


def kernel(x, seg, tok_table, pos_table, seg_table, gamma, beta):
    b, s = x.shape
    v, d = tok_table.shape

    comb_a = pos_table + seg_table[0][None]
    comb_d = jnp.broadcast_to(seg_table[1] - seg_table[0], (s, d))
    g8 = jnp.broadcast_to(gamma, (8, d))
    b8 = jnp.broadcast_to(beta, (8, d))

    bs = b // _NSLICE
    n_chunks = bs * s // _CW
    per_w = n_chunks // _NW
    gather = _make_gather(n_chunks, d)
    ln = _make_ln(bs, s, d)

    outs = []
    for i in range(_NSLICE):
        xi = lax.slice_in_dim(x, i * bs, (i + 1) * bs, axis=0)
        xi = xi.reshape(_NW, per_w, _CW).astype(jnp.int32)
        rows = gather(xi, tok_table).reshape(bs, s, d)
        si = lax.slice_in_dim(seg, i * bs, (i + 1) * bs, axis=0)
        outs.append(ln(rows, si.astype(jnp.float32), comb_a, comb_d,
                       g8, b8))
    return jnp.concatenate(outs, axis=0)


# speedup vs baseline: 9.2128x; 1.1753x over previous
import functools

import jax
import jax.numpy as jnp
from jax import lax
from jax.experimental import pallas as pl
from jax.experimental.pallas import tpu as pltpu
from jax.experimental.pallas import tpu_sc as plsc

_NC = 2
_NS = 16
_NW = _NC * _NS
_L = 16

_CW = 128
_NBUF = 3
_EPS = 1e-5
_NSLICE = 2
_RB = 16


def _make_gather(n_chunks, d_model):
    per_w = n_chunks // _NW

    mesh = plsc.VectorSubcoreMesh(core_axis_name="c", subcore_axis_name="s")

    @functools.partial(
        pl.kernel,
        out_type=jax.ShapeDtypeStruct((_NW, per_w, _CW, d_model),
                                      jnp.float32),
        mesh=mesh,
        scratch_types=[
            pltpu.VMEM((per_w, _CW), jnp.int32),
            pltpu.VMEM((_CW, d_model), jnp.float32),
            pltpu.VMEM((_CW, d_model), jnp.float32),
            pltpu.VMEM((_CW, d_model), jnp.float32),
            pltpu.SemaphoreType.DMA,
            pltpu.SemaphoreType.DMA,
            pltpu.SemaphoreType.DMA,
            pltpu.SemaphoreType.DMA,
            pltpu.SemaphoreType.DMA,
            pltpu.SemaphoreType.DMA,
        ],
    )
    def gather_kernel(x_hbm, tok_hbm, out_hbm, idx_v,
                      rows0, rows1, rows2, gs0, gs1, gs2, os0, os1, os2):
        rows = (rows0, rows1, rows2)
        gsem = (gs0, gs1, gs2)
        osem = (os0, os1, os2)
        wid = lax.axis_index("s") * _NC + lax.axis_index("c")

        pltpu.sync_copy(x_hbm.at[wid], idx_v)

        def start_gather(c, b):
            pltpu.async_copy(tok_hbm.at[idx_v.at[c]], rows[b], gsem[b])

        def wait_gather(c, b):
            pltpu.make_async_copy(
                tok_hbm.at[idx_v.at[c]], rows[b], gsem[b]).wait()

        def start_out(c, b):
            pltpu.async_copy(rows[b], out_hbm.at[wid, c], osem[b])

        def wait_out(c, b):
            pltpu.make_async_copy(
                rows[b], out_hbm.at[wid, c], osem[b]).wait()

        start_gather(0, 0)
        start_gather(1, 1)

        def slot(c, b, with_gather):
            wait_gather(c, b)
            start_out(c, b)
            if with_gather:
                nb = (b + 2) % _NBUF

                @pl.when(c >= 1)
                def _():
                    wait_out(c - 1, nb)

                start_gather(c + 2, nb)

        def pipe_body(t, _):
            c = t * _NBUF
            for r in range(_NBUF):
                slot(c + r, r, True)
            return ()

        n_full = (per_w - 2) // _NBUF
        lax.fori_loop(0, n_full, pipe_body, (), unroll=False)
        for c in range(_NBUF * n_full, per_w):
            slot(c, c % _NBUF, c <= per_w - 3)
        for c in (per_w - 3, per_w - 2, per_w - 1):
            wait_out(c, c % _NBUF)

    return gather_kernel


def _ln_block(tok_ref, seg_ref, comb_a_ref, comb_d_ref, g_ref, b_ref,
              w_ref, out_ref):
    t = tok_ref[...]
    s = seg_ref[...][..., None]
    h = t + comb_a_ref[...][None] + s * comb_d_ref[...][None]
    d = t.shape[-1]
    h2 = h.reshape(-1, d)
    w = w_ref[...]
    mean = jnp.dot(h2, w, preferred_element_type=jnp.float32)
    msq = jnp.dot(h2 * h2, w, preferred_element_type=jnp.float32)
    inv = lax.rsqrt(msq - mean * mean + _EPS)
    norm = (h2 - mean) * inv
    out2 = norm * g_ref[...][0][None] + b_ref[...][0][None]
    out_ref[...] = out2.reshape(t.shape)


def _make_ln(nb, seq, d_model):
    grid = (nb // _RB,)
    return pl.pallas_call(
        _ln_block,
        grid=grid,
        in_specs=[
            pl.BlockSpec((_RB, seq, d_model), lambda i: (i, 0, 0)),
            pl.BlockSpec((_RB, seq), lambda i: (i, 0)),
            pl.BlockSpec((seq, d_model), lambda i: (0, 0)),
            pl.BlockSpec((seq, d_model), lambda i: (0, 0)),
            pl.BlockSpec((8, d_model), lambda i: (0, 0)),
            pl.BlockSpec((8, d_model), lambda i: (0, 0)),
            pl.BlockSpec((d_model, d_model), lambda i: (0, 0)),
        ],
        out_specs=pl.BlockSpec((_RB, seq, d_model), lambda i: (i, 0, 0)),
        out_shape=jax.ShapeDtypeStruct((nb, seq, d_model), jnp.float32),
    )


def kernel(x, seg, tok_table, pos_table, seg_table, gamma, beta):
    b, s = x.shape
    v, d = tok_table.shape

    comb_a = pos_table + seg_table[0][None]
    comb_d = jnp.broadcast_to(seg_table[1] - seg_table[0], (s, d))
    g8 = jnp.broadcast_to(gamma, (8, d))
    b8 = jnp.broadcast_to(beta, (8, d))
    wmean = jnp.full((d, d), 1.0 / d, jnp.float32)

    bs = b // _NSLICE
    n_chunks = bs * s // _CW
    per_w = n_chunks // _NW
    gather = _make_gather(n_chunks, d)
    ln = _make_ln(bs, s, d)

    outs = []
    for i in range(_NSLICE):
        xi = lax.slice_in_dim(x, i * bs, (i + 1) * bs, axis=0)
        xi = xi.reshape(_NW, per_w, _CW).astype(jnp.int32)
        rows = gather(xi, tok_table).reshape(bs, s, d)
        si = lax.slice_in_dim(seg, i * bs, (i + 1) * bs, axis=0)
        outs.append(ln(rows, si.astype(jnp.float32), comb_a, comb_d,
                       g8, b8, wmean))
    return jnp.concatenate(outs, axis=0)
